# Initial kernel scaffold; baseline (speedup 1.0000x reference)
import jax
import jax.numpy as jnp
from jax.experimental import pallas as pl


def kernel(x, edge_index, batch, emb, W1, a1s, a1d, b1, ln1g, ln1b, W2, a2s, a2d, b2, ln2g, ln2b, W3, a3s, a3d, b3, ln3g, ln3b, fc1w, fc1b, fc2w, fc2b):
    raise NotImplementedError("write your pallas kernel here")


# speedup vs baseline: 16.7426x; 16.7426x over previous
import functools

import jax
import jax.numpy as jnp
from jax import lax
from jax.experimental import pallas as pl
from jax.experimental.pallas import tpu as pltpu
from jax.experimental.pallas import tpu_sc as plsc

_N = 10000
_E = 160000
_EP = _E + _N
_G = 16

_NTILES = 32
_B2 = 128
_NB2 = 42
_EPT = _B2 * _NB2
_EPAD = _NTILES * _EPT
_B1 = 256
_RPT = _N // 16

_i32 = jnp.int32
_f32 = jnp.float32


def _make_phase1(heads, hpt):
    eg_n = _NTILES // (heads // hpt)
    epg = _EPAD // eg_n
    nb = epg // _B1
    mesh = plsc.VectorSubcoreMesh(core_axis_name="c", subcore_axis_name="s")

    @functools.partial(
        pl.kernel,
        out_type=(
            jax.ShapeDtypeStruct((heads, _EPAD), _f32),
            jax.ShapeDtypeStruct((eg_n, heads, _N), _f32),
        ),
        mesh=mesh,
        compiler_params=pltpu.CompilerParams(needs_layout_passes=False, use_tc_tiling_on_sc=False),
        scratch_types=[
            pltpu.VMEM((hpt, _N), _f32),
            pltpu.VMEM((hpt, _N), _f32),
            pltpu.VMEM((hpt, _N), _f32),
            pltpu.VMEM((hpt, 16), _f32),
            pltpu.VMEM((_B1,), _i32),
            pltpu.VMEM((_B1,), _i32),
            pltpu.VMEM((hpt, _B1), _f32),
        ],
    )
    def phase1(src_hbm, dst_hbm, als_hbm, ald_hbm, m_hbm, ex_hbm, den_hbm,
               als_v, ald_v, den_v, m_v, src_v, dst_v, ex_v):
        wid = lax.axis_index("s") * 2 + lax.axis_index("c")
        hg = wid // eg_n
        eg = wid % eg_n

        pltpu.sync_copy(als_hbm.at[pl.ds(hg * hpt, hpt)], als_v)
        pltpu.sync_copy(ald_hbm.at[pl.ds(hg * hpt, hpt)], ald_v)
        pltpu.sync_copy(m_hbm.at[pl.ds(hg * hpt, hpt)], m_v)

        zero = jnp.zeros((16,), _f32)

        def zbody(i, _):
            for h in range(hpt):
                den_v[h, pl.ds(i * 16, 16)] = zero
            return 0
        lax.fori_loop(0, _N // 16, zbody, 0)

        mh = [m_v[h, pl.ds(0, 16)] for h in range(hpt)]
        lanes = lax.iota(_i32, 16)
        e_base = eg * epg

        def body(b, _):
            off = e_base + b * _B1
            pltpu.sync_copy(src_hbm.at[pl.ds(off, _B1)], src_v)
            pltpu.sync_copy(dst_hbm.at[pl.ds(off, _B1)], dst_v)

            def grp(g, _):
                s_idx = src_v[pl.ds(g * 16, 16)]
                d_idx = dst_v[pl.ds(g * 16, 16)]
                valid = (off + g * 16 + lanes) < _EP
                for h in range(hpt):
                    hfull = jnp.full((16,), h, _i32)
                    av = plsc.load_gather(als_v, [hfull, s_idx])
                    bv = plsc.load_gather(ald_v, [hfull, d_idx])
                    x = av + bv
                    y = jnp.maximum(x, 0.2 * x)
                    ex = jnp.exp(y - mh[h])
                    ex = jnp.where(valid, ex, 0.0)
                    ex_v[h, pl.ds(g * 16, 16)] = ex
                    plsc.addupdate_scatter(den_v, [hfull, d_idx], ex)
                return 0
            lax.fori_loop(0, _B1 // 16, grp, 0)
            for h in range(hpt):
                pltpu.sync_copy(ex_v.at[h],
                                ex_hbm.at[hg * hpt + h].at[pl.ds(off, _B1)])
            return 0
        lax.fori_loop(0, nb, body, 0)

        pltpu.sync_copy(den_v, den_hbm.at[eg].at[pl.ds(hg * hpt, hpt)])

    return phase1


def _make_phase2(heads, ch, n_chunks):
    cw = heads * ch // n_chunks
    assert ch % cw == 0 or cw % ch == 0
    zr = 125
    mesh = plsc.VectorSubcoreMesh(core_axis_name="c", subcore_axis_name="s")

    @functools.partial(
        pl.kernel,
        out_type=jax.ShapeDtypeStruct((2, n_chunks, _N, cw), _f32),
        mesh=mesh,
        compiler_params=pltpu.CompilerParams(needs_layout_passes=False, use_tc_tiling_on_sc=False),
        scratch_types=[
            pltpu.VMEM((_NB2, _B2), _i32),
            pltpu.VMEM((_NB2, _B2), _i32),
            pltpu.VMEM((1, _EPT), _f32),
            pltpu.VMEM((_B2, cw), _f32),
            pltpu.VMEM((_B2, cw), _f32),
            pltpu.VMEM((zr, cw), _f32),
            pltpu.VMEM_SHARED((_N, cw), _f32),
            pltpu.SemaphoreType.DMA,
            pltpu.SemaphoreType.DMA,
        ],
    )
    def phase2(src_hbm, dst_hbm, ex_hbm, t_hbm, out_hbm,
               src_v, dst_v, ex_v, buf0, buf1, zbuf, acc, sem0, sem1):
        sc = lax.axis_index("c")
        sid = lax.axis_index("s")
        wid = sid * 2 + sc

        pltpu.sync_copy(src_hbm.at[wid], src_v)
        pltpu.sync_copy(dst_hbm.at[wid], dst_v)

        zero = jnp.zeros((16,), _f32)

        def zb(i, _):
            for j in range(cw // 16):
                zbuf[i, pl.ds(j * 16, 16)] = zero
            return 0
        lax.fori_loop(0, zr, zb, 0)

        bufs = (buf0, buf1)
        sems = (sem0, sem1)

        for c in range(n_chunks):
            hd = c * cw // ch
            pltpu.sync_copy(
                ex_hbm.at[hd].at[pl.ds(wid * _EPT, _EPT)], ex_v.at[0])

            for s in range(_RPT // zr):
                pltpu.sync_copy(
                    zbuf, acc.at[pl.ds(sid * _RPT + s * zr, zr)])
            plsc.subcore_barrier()

            table = t_hbm.at[c]
            pltpu.async_copy(table.at[src_v.at[0]], bufs[0], sems[0])

            def body(i, _):
                for q in range(2):
                    b = i * 2 + q
                    buf = bufs[q]
                    sem = sems[q]
                    pltpu.make_async_copy(table.at[src_v.at[b]], buf,
                                          sem).wait()

                    @pl.when(b + 1 < _NB2)
                    def _():
                        pltpu.async_copy(table.at[src_v.at[b + 1]],
                                         bufs[1 - q], sems[1 - q])

                    def pe(e, _):
                        col = b * _B2 + e
                        cfull = jnp.full((16,), col, _i32)
                        w = plsc.load_gather(
                            ex_v, [jnp.zeros((16,), _i32), cfull])
                        for j in range(cw // 16):
                            buf[e, pl.ds(j * 16, 16)] = (
                                buf[e, pl.ds(j * 16, 16)] * w)
                        return 0
                    lax.fori_loop(0, _B2, pe, 0)

                    pltpu.sync_copy(buf, acc.at[dst_v.at[b]], add=True)
                return 0
            lax.fori_loop(0, _NB2 // 2, body, 0)

            plsc.subcore_barrier()
            pltpu.sync_copy(
                acc.at[pl.ds(sid * _RPT, _RPT)],
                out_hbm.at[sc].at[c].at[pl.ds(sid * _RPT, _RPT)])
            plsc.subcore_barrier()

    return phase2


_RB = 1000
_NRB = _N // _RB


def _embed_body(x_ref, emb_ref, h_ref):
    x = x_ref[...]
    iota = lax.broadcasted_iota(_i32, (_RB, 32), 1)
    onehot = (iota == x).astype(_f32)
    h_ref[...] = jnp.dot(onehot, emb_ref[...],
                         preferred_element_type=_f32)


def _embed(x2d, emb):
    return pl.pallas_call(
        _embed_body,
        grid=(_NRB,),
        in_specs=[
            pl.BlockSpec((_RB, 1), lambda r: (r, 0)),
            pl.BlockSpec((32, 16), lambda r: (0, 0)),
        ],
        out_specs=pl.BlockSpec((_RB, 16), lambda r: (r, 0)),
        out_shape=jax.ShapeDtypeStruct((_N, 16), _f32),
    )(x2d, emb)


def _make_dense_pre(fin, heads, ch, n_chunks):
    fout = heads * ch
    cw = fout // n_chunks

    def body(h_ref, w_ref, as_ref, ad_ref, t_ref, als_ref, ald_ref, m_ref,
             macc):
        r = pl.program_id(0)
        t = jnp.dot(h_ref[...], w_ref[...], preferred_element_type=_f32)
        th = t.reshape(_RB, heads, ch)
        als = jnp.sum(th * as_ref[...][None], axis=-1)
        ald = jnp.sum(th * ad_ref[...][None], axis=-1)
        t_ref[...] = t.reshape(_RB, n_chunks, cw).transpose(1, 0, 2)
        als_ref[...] = als
        ald_ref[...] = ald

        pad = jnp.full((1, 16 - heads), -1e30, _f32)
        amax = jnp.concatenate([jnp.max(als, axis=0)[None], pad], axis=1)
        dmax = jnp.concatenate([jnp.max(ald, axis=0)[None], pad], axis=1)

        @pl.when(r == 0)
        def _():
            macc[...] = jnp.full((2, 16), -1e30, _f32)

        macc[0:1] = jnp.maximum(macc[0:1], amax)
        macc[1:2] = jnp.maximum(macc[1:2], dmax)

        @pl.when(r == _NRB - 1)
        def _():
            m_ref[...] = macc[0:1] + macc[1:2]

    def run(h, w, a_s, a_d):
        return pl.pallas_call(
            body,
            grid=(_NRB,),
            in_specs=[
                pl.BlockSpec((_RB, fin), lambda r: (r, 0)),
                pl.BlockSpec((fin, fout), lambda r: (0, 0)),
                pl.BlockSpec((heads, ch), lambda r: (0, 0)),
                pl.BlockSpec((heads, ch), lambda r: (0, 0)),
            ],
            out_specs=[
                pl.BlockSpec((n_chunks, _RB, cw), lambda r: (0, r, 0)),
                pl.BlockSpec((_RB, heads), lambda r: (r, 0)),
                pl.BlockSpec((_RB, heads), lambda r: (r, 0)),
                pl.BlockSpec((1, 16), lambda r: (0, 0)),
            ],
            out_shape=[
                jax.ShapeDtypeStruct((n_chunks, _N, cw), _f32),
                jax.ShapeDtypeStruct((_N, heads), _f32),
                jax.ShapeDtypeStruct((_N, heads), _f32),
                jax.ShapeDtypeStruct((1, 16), _f32),
            ],
            scratch_shapes=[pltpu.VMEM((2, 16), _f32)],
        )(h, w, a_s, a_d)

    return run


_RBP = 400
_NRBP = _N // _RBP


def _make_dense_post(heads, ch, n_chunks, eg_n, mean_heads):
    fout = heads * ch
    cw = fout // n_chunks
    fres = ch if mean_heads else fout

    def body(acc_ref, den_ref, b_ref, g_ref, bn_ref, o_ref):
        s = acc_ref[0] + acc_ref[1]
        s = s.transpose(1, 0, 2).reshape(_RBP, fout)
        den = jnp.sum(den_ref[...], axis=1)
        expand = (lax.broadcasted_iota(_i32, (heads, fout), 1) // ch ==
                  lax.broadcasted_iota(_i32, (heads, fout), 0)).astype(_f32)
        dexp = jnp.dot(den, expand,
                       preferred_element_type=_f32)
        out = s / (dexp + 1e-16)
        if mean_heads:
            out = out.reshape(_RBP, heads, ch).mean(axis=1)
        out = out + b_ref[...]
        mu = jnp.mean(out, axis=-1, keepdims=True)
        var = jnp.mean((out - mu) ** 2, axis=-1, keepdims=True)
        xn = (out - mu) / jnp.sqrt(var + 1e-5) * g_ref[...] + bn_ref[...]
        o_ref[...] = jnp.where(xn > 0, xn, jnp.exp(jnp.minimum(xn, 0.)) - 1.)

    def run(accp, denp_t, b, lng, lnb):
        return pl.pallas_call(
            body,
            grid=(_NRBP,),
            in_specs=[
                pl.BlockSpec((2, n_chunks, _RBP, cw), lambda r: (0, 0, r, 0)),
                pl.BlockSpec((_RBP, eg_n, heads), lambda r: (r, 0, 0)),
                pl.BlockSpec((1, fres), lambda r: (0, 0)),
                pl.BlockSpec((1, fres), lambda r: (0, 0)),
                pl.BlockSpec((1, fres), lambda r: (0, 0)),
            ],
            out_specs=pl.BlockSpec((_RBP, fres), lambda r: (r, 0)),
            out_shape=jax.ShapeDtypeStruct((_N, fres), _f32),
        )(accp, denp_t, b.reshape(1, fres), lng.reshape(1, fres),
          lnb.reshape(1, fres))

    return run


def _final_body(h_ref, bat_ref, w1_ref, b1_ref, w2_ref, b2_ref, o_ref):
    iota = lax.broadcasted_iota(_i32, (_N, _G), 1)
    onehot = (iota == bat_ref[...]).astype(_f32)
    sums = lax.dot_general(onehot, h_ref[...], (((0,), (0,)), ((), ())),
                           preferred_element_type=_f32)
    cnt = jnp.sum(onehot, axis=0)[:, None]
    pooled = sums / jnp.maximum(cnt, 1.0)
    z = jnp.dot(pooled, w1_ref[...], preferred_element_type=_f32) + b1_ref[...]
    z = jnp.maximum(z, 0.0)
    o_ref[...] = (jnp.dot(z, w2_ref[...], preferred_element_type=_f32)
                  + b2_ref[...])


def _final(h3, bat2d, fc1w, fc1b, fc2w, fc2b):
    return pl.pallas_call(
        _final_body,
        grid=(1,),
        in_specs=[
            pl.BlockSpec((_N, 16), lambda i: (0, 0)),
            pl.BlockSpec((_N, 1), lambda i: (0, 0)),
            pl.BlockSpec((16, 16), lambda i: (0, 0)),
            pl.BlockSpec((1, 16), lambda i: (0, 0)),
            pl.BlockSpec((16, 8), lambda i: (0, 0)),
            pl.BlockSpec((1, 8), lambda i: (0, 0)),
        ],
        out_specs=pl.BlockSpec((_G, 8), lambda i: (0, 0)),
        out_shape=jax.ShapeDtypeStruct((_G, 8), _f32),
    )(h3, bat2d, fc1w, fc1b.reshape(1, 16), fc2w, fc2b.reshape(1, 8))


_L1 = (8, 64, 8, 2)
_L2 = (4, 128, 8, 1)
_L3 = (1, 16, 1, 1)

_p1_cache = {}
_p2_cache = {}


def _get_p1(cfg):
    if cfg not in _p1_cache:
        _p1_cache[cfg] = _make_phase1(cfg[0], cfg[3])
    return _p1_cache[cfg]


def _get_p2(cfg):
    if cfg not in _p2_cache:
        _p2_cache[cfg] = _make_phase2(cfg[0], cfg[1], cfg[2])
    return _p2_cache[cfg]


def _gat_layer(cfg, h, fin, src3, dst3, src1, dst1, W, a_s, a_d, b, lng, lnb,
               mean_heads):
    heads, ch, n_chunks, hpt = cfg
    eg_n = _NTILES // (heads // hpt)
    pre = _make_dense_pre(fin, heads, ch, n_chunks)
    t_chunks, als, ald, m = pre(h, W, a_s, a_d)
    alsT = jnp.asarray(als.T)
    aldT = jnp.asarray(ald.T)
    m_rows = jnp.broadcast_to(m.reshape(16, 1), (16, 16))
    ex, denp = _get_p1(cfg)(src1, dst1, alsT, aldT, m_rows)
    accp = _get_p2(cfg)(src3, dst3, ex, t_chunks)
    post = _make_dense_post(heads, ch, n_chunks, eg_n, mean_heads)
    return post(accp, jnp.transpose(denp, (2, 0, 1)), b, lng, lnb)


def kernel(x, edge_index, batch, emb, W1, a1s, a1d, b1, ln1g, ln1b,
           W2, a2s, a2d, b2, ln2g, ln2b, W3, a3s, a3d, b3, ln3g, ln3b,
           fc1w, fc1b, fc2w, fc2b):
    loop = jnp.arange(_N, dtype=_i32)
    src = jnp.concatenate([edge_index[0].astype(_i32), loop])
    dst = jnp.concatenate([edge_index[1].astype(_i32), loop])
    pad = jnp.zeros((_EPAD - _EP,), _i32)
    src1 = jnp.concatenate([src, pad])
    dst1 = jnp.concatenate([dst, pad])
    src3 = src1.reshape(_NTILES, _NB2, _B2)
    dst3 = dst1.reshape(_NTILES, _NB2, _B2)

    h = _embed(x.reshape(_N, 1).astype(_i32), emb)
    h = _gat_layer(_L1, h, 16, src3, dst3, src1, dst1,
                   W1, a1s, a1d, b1, ln1g, ln1b, False)
    h = _gat_layer(_L2, h, 512, src3, dst3, src1, dst1,
                   W2, a2s, a2d, b2, ln2g, ln2b, False)
    h = _gat_layer(_L3, h, 512, src3, dst3, src1, dst1,
                   W3, a3s, a3d, b3, ln3g, ln3b, True)
    return _final(h, batch.reshape(_N, 1).astype(_i32), fc1w, fc1b,
                  fc2w, fc2b)
